# baseline (speedup 1.0000x reference)
import jax
import jax.numpy as jnp
from jax import lax
from jax.experimental import pallas as pl
from jax.experimental.pallas import tpu as pltpu
from jax.experimental.pallas import tpu_sc as plsc

B = 8192
L = 2048
TW = 64
PW = L - TW
NW = 32
RPW = B // NW
GK = 16
SENT = 3.4e38


def _sc_body(t_hbm, tpad_hbm, tails_hbm, out_hbm,
             tt, tvec, obuf, idxbuf, rowg, sem, sem2):
    wid = lax.axis_index("s") * 2 + lax.axis_index("c")
    base = wid * RPW
    cp_tails = pltpu.async_copy(tails_hbm.at[wid], tt, sem)
    cp_t = pltpu.async_copy(t_hbm.at[pl.ds(base, RPW)], tvec, sem2)
    lane = lax.iota(jnp.int32, 16)
    zeros = jnp.zeros((16,), jnp.float32)
    negs = jnp.full((16,), -3.4e38, jnp.float32)
    neg1 = jnp.full((16,), -1, jnp.int32)

    def bfly_max(v):
        for k in (1, 2, 4, 8):
            v = jnp.maximum(v, v[lane ^ k])
        return v

    def finish_half(rowbase, tb16, bv, cnt):
        hit = bv < 1.0e38
        obuf[pl.ds(rowbase, 16)] = jnp.where(hit, jnp.exp(-(tb16 - bv)), zeros)
        miss = jnp.where(hit, 0, 1)
        s = miss
        for k in (1, 2, 4, 8):
            s = s + s[lane ^ k]
        total = s[0]

        @pl.when(total > 0)
        def _():
            run = cnt
            for r in range(16):
                mr = miss[r]

                @pl.when(mr > 0)
                def _(run=run, r=r):
                    cur = idxbuf[pl.ds(run, 16)]
                    idxbuf[pl.ds(run, 16)] = jnp.where(
                        lane == 0,
                        jnp.full((16,), base + rowbase + r, jnp.int32), cur)

                run = run + mr

        return cnt + total

    basev = jnp.full((16,), 0, jnp.int32) + base
    for q in range((RPW + 16) // 16):
        idxbuf[pl.ds(q * 16, 16)] = basev
    cp_tails.wait()
    cp_t.wait()

    def pair_body(p, cnt):
        rbA = p * 32
        rbB = rbA + 16
        tbA = tvec[pl.ds(rbA, 16)]
        tbB = tvec[pl.ds(rbB, 16)]

        def col8(jo, c):
            bvA, bvB = c
            for ji in range(8):
                xA = tt[jo * 8 + ji, pl.ds(rbA, 16)]
                xB = tt[jo * 8 + ji, pl.ds(rbB, 16)]
                bvA = jnp.where(xA <= tbA, xA, bvA)
                bvB = jnp.where(xB <= tbB, xB, bvB)
            return bvA, bvB

        sent = jnp.full((16,), SENT, jnp.float32)
        bvA, bvB = lax.fori_loop(0, TW // 8, col8, (sent, sent))
        cnt = finish_half(rbA, tbA, bvA, cnt)
        cnt = finish_half(rbB, tbB, bvB, cnt)
        return cnt

    kmiss = lax.fori_loop(0, RPW // 32, pair_body, jnp.int32(0))

    def scan_miss_row(j, bi):
        loc = idxbuf[pl.ds(bi * GK + j, 16)][0] - base
        q16v = (loc // 16) * 16
        sel = lane == (loc - q16v)
        tbv = bfly_max(jnp.where(sel, tvec[pl.ds(q16v, 16)], negs))

        def chunk8(ko, c2):
            bi2, bv2 = c2
            for ki in range(8):
                k = ko * 8 + ki
                x = rowg[j, pl.ds(k * 16, 16)]
                c = x <= tbv
                bi2 = jnp.where(c, jnp.full((16,), k, jnp.int32), bi2)
                bv2 = jnp.where(c, x, bv2)
            return bi2, bv2

        bi2, bv2 = lax.fori_loop(0, L // 128, chunk8, (neg1, zeros))
        g = jnp.where(bi2 >= 0, bi2 * 16 + lane, neg1)
        bv = bv2
        for k in (1, 2, 4, 8):
            og = g[lane ^ k]
            ob = bv[lane ^ k]
            take = og > g
            g = jnp.where(take, og, g)
            bv = jnp.where(take, ob, bv)
        res16 = jnp.where(g >= 0, jnp.exp(-(tbv - bv)), zeros)
        cur = obuf[pl.ds(q16v, 16)]
        obuf[pl.ds(q16v, 16)] = jnp.where(sel, res16, cur)

    @pl.when(kmiss > 0)
    def _():
        def batch_body(bi, carry):
            pltpu.async_copy(
                tpad_hbm.at[idxbuf.at[pl.ds(bi * GK, GK)]], rowg, sem).wait()

            def row_j(j, c2):
                pl.when(bi * GK + j < kmiss)(lambda: scan_miss_row(j, bi))
                return c2

            lax.fori_loop(0, GK, row_j, 0)
            return carry

        nb = (kmiss + (GK - 1)) // GK
        lax.fori_loop(0, nb, batch_body, 0)

    pltpu.sync_copy(obuf, out_hbm.at[pl.ds(base, RPW)])


@jax.jit
def _sc_call(t, t_pad, tails):
    mesh = plsc.VectorSubcoreMesh(core_axis_name="c", subcore_axis_name="s")
    f = pl.kernel(
        _sc_body,
        mesh=mesh,
        out_type=jax.ShapeDtypeStruct((B,), jnp.float32),
        scratch_types=[
            pltpu.VMEM((TW, RPW), jnp.float32),
            pltpu.VMEM((RPW,), jnp.float32),
            pltpu.VMEM((RPW,), jnp.float32),
            pltpu.VMEM((RPW + 16,), jnp.int32),
            pltpu.VMEM((GK, L), jnp.float32),
            pltpu.SemaphoreType.DMA,
            pltpu.SemaphoreType.DMA,
        ],
    )
    return f(t, t_pad, tails)


def _tails_body(tp_ref, o_ref):
    o_ref[...] = jnp.transpose(tp_ref[...])[None, 128 - TW:, :]


def _tails_tc(t_pad):
    return pl.pallas_call(
        _tails_body,
        grid=(NW,),
        in_specs=[pl.BlockSpec((RPW, 128), lambda i: (i, L // 128 - 1))],
        out_specs=pl.BlockSpec((1, TW, RPW), lambda i: (i, 0, 0)),
        out_shape=jax.ShapeDtypeStruct((NW, TW, RPW), jnp.float32),
    )(t_pad)


def kernel(src, dst, t, x_pad_simu, t_pad, emb_src, emb_dst):
    return _sc_call(t, t_pad, _tails_tc(t_pad))


# speedup vs baseline: 1.4176x; 1.4176x over previous
import jax
import jax.numpy as jnp
from jax import lax
from jax.experimental import pallas as pl
from jax.experimental.pallas import tpu as pltpu
from jax.experimental.pallas import tpu_sc as plsc

B = 8192
L = 2048
TW = 64
PW = L - TW
NW = 32
RPW = B // NW
GK = 16
SENT = 3.4e38


def _sc_body(t_hbm, tpad_hbm, tails_hbm, out_hbm,
             tt, tvec, obuf, idxbuf, rowg, sem, sem2):
    wid = lax.axis_index("s") * 2 + lax.axis_index("c")
    base = wid * RPW
    cp_tails = pltpu.async_copy(tails_hbm.at[wid], tt, sem)
    cp_t = pltpu.async_copy(t_hbm.at[pl.ds(base, RPW)], tvec, sem2)
    lane = lax.iota(jnp.int32, 16)
    zeros = jnp.zeros((16,), jnp.float32)
    negs = jnp.full((16,), -3.4e38, jnp.float32)
    neg1 = jnp.full((16,), -1, jnp.int32)

    def bfly_max(v):
        for k in (1, 2, 4, 8):
            v = jnp.maximum(v, v[lane ^ k])
        return v

    def finish_half(rowbase, tb16, bv, cnt):
        hit = bv < 1.0e38
        obuf[pl.ds(rowbase, 16)] = jnp.where(hit, jnp.exp(-(tb16 - bv)), zeros)
        miss = jnp.where(hit, 0, 1)
        s = miss
        for k in (1, 2, 4, 8):
            s = s + s[lane ^ k]
        total = s[0]

        @pl.when(total > 0)
        def _():
            run = cnt
            for r in range(16):
                mr = miss[r]

                @pl.when(mr > 0)
                def _(run=run, r=r):
                    cur = idxbuf[pl.ds(run, 16)]
                    idxbuf[pl.ds(run, 16)] = jnp.where(
                        lane == 0,
                        jnp.full((16,), base + rowbase + r, jnp.int32), cur)

                run = run + mr

        return cnt + total

    basev = jnp.full((16,), 0, jnp.int32) + base
    for q in range((RPW + 16) // 16):
        idxbuf[pl.ds(q * 16, 16)] = basev
    cp_tails.wait()
    cp_t.wait()

    def pair_body(p, cnt):
        rbA = p * 32
        rbB = rbA + 16
        tbA = tvec[pl.ds(rbA, 16)]
        tbB = tvec[pl.ds(rbB, 16)]

        def col8(jo, c):
            bvA, bvB = c
            for ji in range(8):
                xA = tt[jo * 8 + ji, pl.ds(rbA, 16)]
                xB = tt[jo * 8 + ji, pl.ds(rbB, 16)]
                bvA = jnp.where(xA <= tbA, xA, bvA)
                bvB = jnp.where(xB <= tbB, xB, bvB)
            return bvA, bvB

        sent = jnp.full((16,), SENT, jnp.float32)
        bvA, bvB = lax.fori_loop(0, TW // 8, col8, (sent, sent))
        cnt = finish_half(rbA, tbA, bvA, cnt)
        cnt = finish_half(rbB, tbB, bvB, cnt)
        return cnt

    kmiss = lax.fori_loop(0, RPW // 32, pair_body, jnp.int32(0))

    def scan_miss_row(j, bi):
        loc = idxbuf[pl.ds(bi * GK + j, 16)][0] - base
        q16v = (loc // 16) * 16
        sel = lane == (loc - q16v)
        tbv = bfly_max(jnp.where(sel, tvec[pl.ds(q16v, 16)], negs))

        def chunk8(ko, c2):
            bi2, bv2 = c2
            for ki in range(8):
                k = ko * 8 + ki
                x = rowg[j, pl.ds(k * 16, 16)]
                c = x <= tbv
                bi2 = jnp.where(c, jnp.full((16,), k, jnp.int32), bi2)
                bv2 = jnp.where(c, x, bv2)
            return bi2, bv2

        bi2, bv2 = lax.fori_loop(0, L // 128, chunk8, (neg1, zeros))
        g = jnp.where(bi2 >= 0, bi2 * 16 + lane, neg1)
        bv = bv2
        for k in (1, 2, 4, 8):
            og = g[lane ^ k]
            ob = bv[lane ^ k]
            take = og > g
            g = jnp.where(take, og, g)
            bv = jnp.where(take, ob, bv)
        res16 = jnp.where(g >= 0, jnp.exp(-(tbv - bv)), zeros)
        cur = obuf[pl.ds(q16v, 16)]
        obuf[pl.ds(q16v, 16)] = jnp.where(sel, res16, cur)

    @pl.when(kmiss > 0)
    def _():
        def batch_body(bi, carry):
            pltpu.async_copy(
                tpad_hbm.at[idxbuf.at[pl.ds(bi * GK, GK)]], rowg, sem).wait()

            def row_j(j, c2):
                pl.when(bi * GK + j < kmiss)(lambda: scan_miss_row(j, bi))
                return c2

            lax.fori_loop(0, GK, row_j, 0)
            return carry

        nb = (kmiss + (GK - 1)) // GK
        lax.fori_loop(0, nb, batch_body, 0)

    pltpu.sync_copy(obuf, out_hbm.at[pl.ds(base, RPW)])


@jax.jit
def _sc_call(t, t_pad, tails):
    mesh = plsc.VectorSubcoreMesh(core_axis_name="c", subcore_axis_name="s")
    f = pl.kernel(
        _sc_body,
        mesh=mesh,
        out_type=jax.ShapeDtypeStruct((B,), jnp.float32),
        scratch_types=[
            pltpu.VMEM((TW, RPW), jnp.float32),
            pltpu.VMEM((RPW,), jnp.float32),
            pltpu.VMEM((RPW,), jnp.float32),
            pltpu.VMEM((RPW + 16,), jnp.int32),
            pltpu.VMEM((GK, L), jnp.float32),
            pltpu.SemaphoreType.DMA,
            pltpu.SemaphoreType.DMA,
        ],
    )
    return f(t, t_pad, tails)


def kernel(src, dst, t, x_pad_simu, t_pad, emb_src, emb_dst):
    tails = jnp.transpose(t_pad[:, PW:].reshape(NW, RPW, TW), (0, 2, 1))
    return _sc_call(t, t_pad, tails)


# speedup vs baseline: 1.4335x; 1.0112x over previous
import jax
import jax.numpy as jnp
from jax import lax
from jax.experimental import pallas as pl
from jax.experimental.pallas import tpu as pltpu
from jax.experimental.pallas import tpu_sc as plsc

B = 8192
L = 2048
TW = 64
PW = L - TW
NW = 32
RPW = B // NW
GK = 16
SENT = 3.4e38


def _sc_body(t_hbm, tpad_hbm, tails_hbm, out_hbm,
             tt, tvec, obuf, idxbuf, rowg, sem, sem2):
    wid = lax.axis_index("s") * 2 + lax.axis_index("c")
    base = wid * RPW
    cp_tails = pltpu.async_copy(tails_hbm.at[wid], tt, sem)
    cp_t = pltpu.async_copy(t_hbm.at[pl.ds(base, RPW)], tvec, sem2)
    lane = lax.iota(jnp.int32, 16)
    zeros = jnp.zeros((16,), jnp.float32)
    negs = jnp.full((16,), -3.4e38, jnp.float32)
    neg1 = jnp.full((16,), -1, jnp.int32)

    def bfly_max(v):
        for k in (1, 2, 4, 8):
            v = jnp.maximum(v, v[lane ^ k])
        return v

    def finish_half(rowbase, tb16, bv, cnt):
        hit = bv < 1.0e38
        obuf[pl.ds(rowbase, 16)] = jnp.where(hit, jnp.exp(-(tb16 - bv)), zeros)
        miss = jnp.where(hit, 0, 1)
        s = miss
        for k in (1, 2, 4, 8):
            s = s + s[lane ^ k]
        total = s[0]

        @pl.when(total > 0)
        def _():
            run = cnt
            for r in range(16):
                mr = miss[r]

                @pl.when(mr > 0)
                def _(run=run, r=r):
                    cur = idxbuf[pl.ds(run, 16)]
                    idxbuf[pl.ds(run, 16)] = jnp.where(
                        lane == 0,
                        jnp.full((16,), base + rowbase + r, jnp.int32), cur)

                run = run + mr

        return cnt + total

    basev = jnp.full((16,), 0, jnp.int32) + base
    for q in range((RPW + 16) // 16):
        idxbuf[pl.ds(q * 16, 16)] = basev
    cp_tails.wait()
    cp_t.wait()

    def pair_body(p, cnt):
        rbA = p * 32
        rbB = rbA + 16
        tbA = tvec[pl.ds(rbA, 16)]
        tbB = tvec[pl.ds(rbB, 16)]

        def col8(jo, c):
            bvA, bvB = c
            for ji in range(16):
                xA = tt[jo * 16 + ji, pl.ds(rbA, 16)]
                xB = tt[jo * 16 + ji, pl.ds(rbB, 16)]
                bvA = jnp.where(xA <= tbA, xA, bvA)
                bvB = jnp.where(xB <= tbB, xB, bvB)
            return bvA, bvB

        sent = jnp.full((16,), SENT, jnp.float32)
        bvA, bvB = lax.fori_loop(0, TW // 16, col8, (sent, sent))
        cnt = finish_half(rbA, tbA, bvA, cnt)
        cnt = finish_half(rbB, tbB, bvB, cnt)
        return cnt

    kmiss = lax.fori_loop(0, RPW // 32, pair_body, jnp.int32(0))

    def scan_miss_row(j, bi):
        loc = idxbuf[pl.ds(bi * GK + j, 16)][0] - base
        q16v = (loc // 16) * 16
        sel = lane == (loc - q16v)
        tbv = bfly_max(jnp.where(sel, tvec[pl.ds(q16v, 16)], negs))

        def chunk8(ko, c2):
            bi2, bv2 = c2
            for ki in range(8):
                k = ko * 8 + ki
                x = rowg[j, pl.ds(k * 16, 16)]
                c = x <= tbv
                bi2 = jnp.where(c, jnp.full((16,), k, jnp.int32), bi2)
                bv2 = jnp.where(c, x, bv2)
            return bi2, bv2

        bi2, bv2 = lax.fori_loop(0, L // 128, chunk8, (neg1, zeros))
        g = jnp.where(bi2 >= 0, bi2 * 16 + lane, neg1)
        bv = bv2
        for k in (1, 2, 4, 8):
            og = g[lane ^ k]
            ob = bv[lane ^ k]
            take = og > g
            g = jnp.where(take, og, g)
            bv = jnp.where(take, ob, bv)
        res16 = jnp.where(g >= 0, jnp.exp(-(tbv - bv)), zeros)
        cur = obuf[pl.ds(q16v, 16)]
        obuf[pl.ds(q16v, 16)] = jnp.where(sel, res16, cur)

    @pl.when(kmiss > 0)
    def _():
        def batch_body(bi, carry):
            pltpu.async_copy(
                tpad_hbm.at[idxbuf.at[pl.ds(bi * GK, GK)]], rowg, sem).wait()

            def row_j(j, c2):
                pl.when(bi * GK + j < kmiss)(lambda: scan_miss_row(j, bi))
                return c2

            lax.fori_loop(0, GK, row_j, 0)
            return carry

        nb = (kmiss + (GK - 1)) // GK
        lax.fori_loop(0, nb, batch_body, 0)

    pltpu.sync_copy(obuf, out_hbm.at[pl.ds(base, RPW)])


@jax.jit
def _sc_call(t, t_pad, tails):
    mesh = plsc.VectorSubcoreMesh(core_axis_name="c", subcore_axis_name="s")
    f = pl.kernel(
        _sc_body,
        mesh=mesh,
        out_type=jax.ShapeDtypeStruct((B,), jnp.float32),
        scratch_types=[
            pltpu.VMEM((TW, RPW), jnp.float32),
            pltpu.VMEM((RPW,), jnp.float32),
            pltpu.VMEM((RPW,), jnp.float32),
            pltpu.VMEM((RPW + 16,), jnp.int32),
            pltpu.VMEM((GK, L), jnp.float32),
            pltpu.SemaphoreType.DMA,
            pltpu.SemaphoreType.DMA,
        ],
    )
    return f(t, t_pad, tails)


def kernel(src, dst, t, x_pad_simu, t_pad, emb_src, emb_dst):
    tails = jnp.transpose(t_pad[:, PW:].reshape(NW, RPW, TW), (0, 2, 1))
    return _sc_call(t, t_pad, tails)
